# baseline (speedup 1.0000x reference)
import functools

import jax
import jax.numpy as jnp
from jax import lax
from jax.experimental import pallas as pl
from jax.experimental.pallas import tpu as pltpu
from jax.experimental.pallas import tpu_sc as plsc

_SCALE = 30.0
_DRW_START = 15
_LN2 = 0.69314718056
_LOG_COEFFS = (
    7.0376836292e-2, -1.1514610310e-1, 1.1676998740e-1, -1.2420140846e-1,
    1.4249322787e-1, -1.6668057665e-1, 2.0000714765e-1, -2.4999993993e-1,
    3.3333331174e-1,
)

_NC = 2
_NS = 16
_NW = _NC * _NS
_L = 16


def _vlog(x):
    xi = plsc.bitcast(x, jnp.int32)
    e = lax.shift_right_arithmetic(xi, 23) - 127
    mi = lax.bitwise_or(lax.bitwise_and(xi, 0x7FFFFF), 0x3F800000)
    mf = plsc.bitcast(mi, jnp.float32)
    big = mf > 1.41421356
    mf = jnp.where(big, mf * 0.5, mf)
    e = e + big.astype(jnp.int32)
    t = mf - 1.0
    p = jnp.full((_L,), _LOG_COEFFS[0], dtype=jnp.float32)
    for c in _LOG_COEFFS[1:]:
        p = p * t + c
    y = t * t * t * p - 0.5 * t * t + t
    return y + e.astype(jnp.float32) * _LN2


def _body(lg_hbm, tg_hbm, ml_hbm, cw_hbm, out_hbm, lg_v, tg_v, ml_v, cw_v,
          out_v, cols_per_w, ncls):
    c = lax.axis_index("c")
    s = lax.axis_index("s")
    wid = s * _NC + c
    col0 = wid * cols_per_w
    pltpu.sync_copy(lg_hbm.at[:, pl.ds(col0, cols_per_w)], lg_v)
    pltpu.sync_copy(tg_hbm.at[pl.ds(col0, cols_per_w)], tg_v)
    pltpu.sync_copy(ml_hbm, ml_v)
    pltpu.sync_copy(cw_hbm, cw_v)

    lane = lax.broadcasted_iota(jnp.int32, (_L,), 0)
    ngroups = cols_per_w // _L

    def groupfn(g, carry):
        awl, aw, an = carry
        boff = g * _L

        def maxfn(cc, m):
            return jnp.maximum(m, lg_v[cc, pl.ds(boff, _L)])

        mg = lax.fori_loop(1, ncls, maxfn, lg_v[0, pl.ds(boff, _L)],
                           unroll=11)

        def sumfn(cc, sacc):
            return sacc + jnp.exp((lg_v[cc, pl.ds(boff, _L)] - mg) * _SCALE)

        sg = lax.fori_loop(0, ncls, sumfn, jnp.zeros((_L,), jnp.float32),
                           unroll=10)
        tgt = tg_v[pl.ds(boff, _L)]
        bvec = boff + lane
        tv = plsc.load_gather(lg_v, [tgt, bvec])
        mm = plsc.load_gather(ml_v, [tgt])
        ww = plsc.load_gather(cw_v, [tgt])
        a = (tv - mg) * _SCALE
        bq = (tv - mm - mg) * _SCALE
        adj = sg - jnp.exp(a) + jnp.exp(bq)
        nll = _vlog(adj) - bq
        return (awl + ww * nll, aw + ww, an + nll)

    zeros = jnp.zeros((_L,), jnp.float32)
    awl, aw, an = lax.fori_loop(0, ngroups, groupfn, (zeros, zeros, zeros))

    perms = [lane ^ sh for sh in (8, 4, 2, 1)]
    gdims = lax.GatherDimensionNumbers(
        offset_dims=(), collapsed_slice_dims=(0,), start_index_map=(0,))

    def _allsum(v):
        for pm in perms:
            v = v + lax.gather(v, pm[:, None], dimension_numbers=gdims,
                               slice_sizes=(1,),
                               mode=lax.GatherScatterMode.PROMISE_IN_BOUNDS)
        return v

    awl = _allsum(awl)
    aw = _allsum(aw)
    an = _allsum(an)
    res = jnp.where(lane == 0, awl,
                    jnp.where(lane == 1, aw,
                              jnp.where(lane == 2, an, 0.0)))
    out_v[...] = res
    pltpu.sync_copy(out_v, out_hbm.at[wid])


def kernel(logits, targets, m_list, class_weights, epoch):
    B, C = logits.shape
    cols_per_w = B // _NW
    targets = targets.astype(jnp.int32)
    logits_t = logits.T

    mesh = plsc.VectorSubcoreMesh(core_axis_name="c", subcore_axis_name="s")
    body = functools.partial(_body, cols_per_w=cols_per_w, ncls=C)
    run = pl.kernel(
        body,
        out_type=jax.ShapeDtypeStruct((_NW, _L), jnp.float32),
        mesh=mesh,
        compiler_params=pltpu.CompilerParams(needs_layout_passes=False),
        scratch_types=[
            pltpu.VMEM((C, cols_per_w), jnp.float32),
            pltpu.VMEM((cols_per_w,), jnp.int32),
            pltpu.VMEM((C,), jnp.float32),
            pltpu.VMEM((C,), jnp.float32),
            pltpu.VMEM((_L,), jnp.float32),
        ],
    )
    part = run(logits_t, targets, m_list, class_weights)
    wl = jnp.sum(part[:, 0])
    wsum = jnp.sum(part[:, 1])
    nsum = jnp.sum(part[:, 2])
    weighted = wl / wsum
    mean = nsum / B
    return jnp.where(epoch >= _DRW_START, weighted, mean)


# speedup vs baseline: 1.0258x; 1.0258x over previous
import functools

import jax
import jax.numpy as jnp
from jax import lax
from jax.experimental import pallas as pl
from jax.experimental.pallas import tpu as pltpu
from jax.experimental.pallas import tpu_sc as plsc

_SCALE = 30.0
_DRW_START = 15
_LN2 = 0.69314718056
_LOG_COEFFS = (
    7.0376836292e-2, -1.1514610310e-1, 1.1676998740e-1, -1.2420140846e-1,
    1.4249322787e-1, -1.6668057665e-1, 2.0000714765e-1, -2.4999993993e-1,
    3.3333331174e-1,
)

_NC = 2
_NS = 16
_NW = _NC * _NS
_L = 16


def _vlog(x):
    xi = plsc.bitcast(x, jnp.int32)
    e = lax.shift_right_arithmetic(xi, 23) - 127
    mi = lax.bitwise_or(lax.bitwise_and(xi, 0x7FFFFF), 0x3F800000)
    mf = plsc.bitcast(mi, jnp.float32)
    big = mf > 1.41421356
    mf = jnp.where(big, mf * 0.5, mf)
    e = e + big.astype(jnp.int32)
    t = mf - 1.0
    p = jnp.full((_L,), _LOG_COEFFS[0], dtype=jnp.float32)
    for c in _LOG_COEFFS[1:]:
        p = p * t + c
    y = t * t * t * p - 0.5 * t * t + t
    return y + e.astype(jnp.float32) * _LN2


def _body(lg_hbm, tg_hbm, ml_hbm, cw_hbm, out_hbm, lg_v, tg_v, ml_v, cw_v,
          out_v, cols_per_w, ncls):
    c = lax.axis_index("c")
    s = lax.axis_index("s")
    wid = s * _NC + c
    col0 = wid * cols_per_w
    pltpu.sync_copy(lg_hbm.at[:, pl.ds(col0, cols_per_w)], lg_v)
    pltpu.sync_copy(tg_hbm.at[pl.ds(col0, cols_per_w)], tg_v)
    pltpu.sync_copy(ml_hbm, ml_v)
    pltpu.sync_copy(cw_hbm, cw_v)

    lane = lax.broadcasted_iota(jnp.int32, (_L,), 0)
    ngroups = cols_per_w // _L

    def groupfn(g, carry):
        awl, aw, an = carry
        boff = g * _L
        mg = lg_v[0, pl.ds(boff, _L)]
        for cc in range(1, ncls):
            mg = jnp.maximum(mg, lg_v[cc, pl.ds(boff, _L)])
        sg = jnp.exp((lg_v[0, pl.ds(boff, _L)] - mg) * _SCALE)
        for cc in range(1, ncls):
            sg = sg + jnp.exp((lg_v[cc, pl.ds(boff, _L)] - mg) * _SCALE)
        tgt = tg_v[pl.ds(boff, _L)]
        bvec = boff + lane
        tv = plsc.load_gather(lg_v, [tgt, bvec])
        mm = plsc.load_gather(ml_v, [tgt])
        ww = plsc.load_gather(cw_v, [tgt])
        a = (tv - mg) * _SCALE
        bq = (tv - mm - mg) * _SCALE
        adj = sg - jnp.exp(a) + jnp.exp(bq)
        nll = _vlog(adj) - bq
        return (awl + ww * nll, aw + ww, an + nll)

    zeros = jnp.zeros((_L,), jnp.float32)
    awl, aw, an = lax.fori_loop(0, ngroups, groupfn, (zeros, zeros, zeros))

    perms = [lane ^ sh for sh in (8, 4, 2, 1)]
    gdims = lax.GatherDimensionNumbers(
        offset_dims=(), collapsed_slice_dims=(0,), start_index_map=(0,))

    def _allsum(v):
        for pm in perms:
            v = v + lax.gather(v, pm[:, None], dimension_numbers=gdims,
                               slice_sizes=(1,),
                               mode=lax.GatherScatterMode.PROMISE_IN_BOUNDS)
        return v

    awl = _allsum(awl)
    aw = _allsum(aw)
    an = _allsum(an)
    res = jnp.where(lane == 0, awl,
                    jnp.where(lane == 1, aw,
                              jnp.where(lane == 2, an, 0.0)))
    out_v[...] = res
    pltpu.sync_copy(out_v, out_hbm.at[wid])


def kernel(logits, targets, m_list, class_weights, epoch):
    B, C = logits.shape
    cols_per_w = B // _NW
    targets = targets.astype(jnp.int32)
    logits_t = logits.T

    mesh = plsc.VectorSubcoreMesh(core_axis_name="c", subcore_axis_name="s")
    body = functools.partial(_body, cols_per_w=cols_per_w, ncls=C)
    run = pl.kernel(
        body,
        out_type=jax.ShapeDtypeStruct((_NW, _L), jnp.float32),
        mesh=mesh,
        compiler_params=pltpu.CompilerParams(needs_layout_passes=False),
        scratch_types=[
            pltpu.VMEM((C, cols_per_w), jnp.float32),
            pltpu.VMEM((cols_per_w,), jnp.int32),
            pltpu.VMEM((C,), jnp.float32),
            pltpu.VMEM((C,), jnp.float32),
            pltpu.VMEM((_L,), jnp.float32),
        ],
    )
    part = run(logits_t, targets, m_list, class_weights)
    wl = jnp.sum(part[:, 0])
    wsum = jnp.sum(part[:, 1])
    nsum = jnp.sum(part[:, 2])
    weighted = wl / wsum
    mean = nsum / B
    return jnp.where(epoch >= _DRW_START, weighted, mean)


# speedup vs baseline: 1.0269x; 1.0011x over previous
import functools

import jax
import jax.numpy as jnp
from jax import lax
from jax.experimental import pallas as pl
from jax.experimental.pallas import tpu as pltpu
from jax.experimental.pallas import tpu_sc as plsc

_SCALE = 30.0
_DRW_START = 15
_LN2 = 0.69314718056
_LOG_COEFFS = (
    7.0376836292e-2, -1.1514610310e-1, 1.1676998740e-1, -1.2420140846e-1,
    1.4249322787e-1, -1.6668057665e-1, 2.0000714765e-1, -2.4999993993e-1,
    3.3333331174e-1,
)

_NC = 2
_NS = 16
_NW = _NC * _NS
_L = 16


def _vlog(x):
    xi = plsc.bitcast(x, jnp.int32)
    e = lax.shift_right_arithmetic(xi, 23) - 127
    mi = lax.bitwise_or(lax.bitwise_and(xi, 0x7FFFFF), 0x3F800000)
    mf = plsc.bitcast(mi, jnp.float32)
    big = mf > 1.41421356
    mf = jnp.where(big, mf * 0.5, mf)
    e = e + big.astype(jnp.int32)
    t = mf - 1.0
    p = jnp.full((_L,), _LOG_COEFFS[0], dtype=jnp.float32)
    for c in _LOG_COEFFS[1:]:
        p = p * t + c
    y = t * t * t * p - 0.5 * t * t + t
    return y + e.astype(jnp.float32) * _LN2


def _body(lg_hbm, tg_hbm, ml_hbm, cw_hbm, out_hbm, lg_v, tg_v, ml_v, cw_v,
          out_v, cols_per_w, ncls):
    c = lax.axis_index("c")
    s = lax.axis_index("s")
    wid = s * _NC + c
    col0 = wid * cols_per_w
    pltpu.sync_copy(lg_hbm.at[:, pl.ds(col0, cols_per_w)], lg_v)
    pltpu.sync_copy(tg_hbm.at[pl.ds(col0, cols_per_w)], tg_v)
    pltpu.sync_copy(ml_hbm, ml_v)
    pltpu.sync_copy(cw_hbm, cw_v)

    lane = lax.broadcasted_iota(jnp.int32, (_L,), 0)
    ngroups = cols_per_w // _L

    def groupfn(g, carry):
        awl, aw, an = carry
        boff = g * _L
        mg = lg_v[0, pl.ds(boff, _L)]
        for cc in range(1, ncls):
            mg = jnp.maximum(mg, lg_v[cc, pl.ds(boff, _L)])
        sg = jnp.exp((lg_v[0, pl.ds(boff, _L)] - mg) * _SCALE)
        for cc in range(1, ncls):
            sg = sg + jnp.exp((lg_v[cc, pl.ds(boff, _L)] - mg) * _SCALE)
        tgt = tg_v[pl.ds(boff, _L)]
        bvec = boff + lane
        tv = plsc.load_gather(lg_v, [tgt, bvec])
        mm = plsc.load_gather(ml_v, [tgt])
        ww = plsc.load_gather(cw_v, [tgt])
        a = (tv - mg) * _SCALE
        bq = (tv - mm - mg) * _SCALE
        adj = sg - jnp.exp(a) + jnp.exp(bq)
        nll = _vlog(adj) - bq
        return (awl + ww * nll, aw + ww, an + nll)

    zeros = jnp.zeros((_L,), jnp.float32)
    awl, aw, an = lax.fori_loop(0, ngroups, groupfn, (zeros, zeros, zeros))

    perms = [lane ^ sh for sh in (8, 4, 2, 1)]
    gdims = lax.GatherDimensionNumbers(
        offset_dims=(), collapsed_slice_dims=(0,), start_index_map=(0,))

    def _allsum(v):
        for pm in perms:
            v = v + lax.gather(v, pm[:, None], dimension_numbers=gdims,
                               slice_sizes=(1,),
                               mode=lax.GatherScatterMode.PROMISE_IN_BOUNDS)
        return v

    awl = _allsum(awl)
    aw = _allsum(aw)
    an = _allsum(an)
    res = jnp.where(lane == 0, awl,
                    jnp.where(lane == 1, aw,
                              jnp.where(lane == 2, an, 0.0)))
    out_v[...] = res
    pltpu.sync_copy(out_v, out_hbm.at[wid])


def kernel(logits, targets, m_list, class_weights, epoch):
    B, C = logits.shape
    cols_per_w = B // _NW
    targets = targets.astype(jnp.int32)
    logits_t = logits.T

    mesh = plsc.VectorSubcoreMesh(core_axis_name="c", subcore_axis_name="s")
    body = functools.partial(_body, cols_per_w=cols_per_w, ncls=C)
    run = pl.kernel(
        body,
        out_type=jax.ShapeDtypeStruct((_NW, _L), jnp.float32),
        mesh=mesh,
        compiler_params=pltpu.CompilerParams(needs_layout_passes=False,
                                             skip_device_barrier=True),
        scratch_types=[
            pltpu.VMEM((C, cols_per_w), jnp.float32),
            pltpu.VMEM((cols_per_w,), jnp.int32),
            pltpu.VMEM((C,), jnp.float32),
            pltpu.VMEM((C,), jnp.float32),
            pltpu.VMEM((_L,), jnp.float32),
        ],
    )
    part = run(logits_t, targets, m_list, class_weights)
    wl = jnp.sum(part[:, 0])
    wsum = jnp.sum(part[:, 1])
    nsum = jnp.sum(part[:, 2])
    weighted = wl / wsum
    mean = nsum / B
    return jnp.where(epoch >= _DRW_START, weighted, mean)
